# baseline (speedup 1.0000x reference)
import functools

import jax
from jax import lax
import jax.numpy as jnp
from jax.experimental import pallas as pl
from jax.experimental.pallas import tpu as pltpu
from jax.experimental.pallas import tpu_sc as plsc

N = 10000
NP = 10240
D = 128
H = 4
C1 = 256
E = 320000
ETOT = E + N
EB = 128
HB = 64
NSUB = 16
NB_K2 = -(-ETOT // (NSUB * EB))
PT_K2 = NB_K2 * EB
NQ = PT_K2 // HB
EP = NSUB * PT_K2
ROWS_T = NP // NSUB
PT_K4 = E // (2 * NSUB)
B_K4 = 1000
HI = lax.Precision.HIGHEST


def _leaky(t):
    return jnp.where(t > 0, t, 0.2 * t)


def _k1_body(x_ref, gW1_ref, gas_ref, gad_ref, o_ref):
    cols = []
    for tbl_ref in (gas_ref, gad_ref):
        for h in range(H):
            blk = gW1_ref[:, h * C1:(h + 1) * C1] * tbl_ref[h:h + 1, :]
            cols.append(jnp.sum(blk, axis=1, keepdims=True))
    vsd = jnp.concatenate(cols, axis=1)
    o_ref[...] = lax.dot_general(vsd, x_ref[...], (((0,), (1,)), ((), ())),
                                 precision=HI)


def _k2_body(a1T, x_hbm, s_hbm, d_hbm, aggw, dpart, agg_sh,
             tbl_s, tbl_d, denom, sA, dA, sB, dB, wA, wB, xrA, xrB,
             gsA, gsB, ssA, ssB):
    core = lax.axis_index("c")
    sub = lax.axis_index("s")

    def scalar_phase(sref, dref, wref, u):
        base = sub * PT_K2 + u * HB
        for i in range(HB // 16):
            s16 = sref[0, pl.ds(i * 16, 16)]
            d16 = dref[0, pl.ds(i * 16, 16)]
            a_s = plsc.load_gather(tbl_s, [s16])
            a_d = plsc.load_gather(tbl_d, [d16])
            w16 = jnp.exp(_leaky(a_s + a_d) - a_d)
            eid = base + i * 16 + lax.iota(jnp.int32, 16)
            w16 = jnp.where(eid < ETOT, w16, 0.0)
            wref[pl.ds(i * 16, 16)] = w16
            plsc.addupdate_scatter(denom, [d16], w16)

    def stage_in(u, sref, dref, xref, gsem, wref):
        off = sub * PT_K2 + u * HB
        pltpu.sync_copy(s_hbm.at[pl.ds(off, HB)], sref.at[0])
        pltpu.sync_copy(d_hbm.at[pl.ds(off, HB)], dref.at[0])
        pltpu.async_copy(x_hbm.at[sref.at[0]], xref, gsem)
        scalar_phase(sref, dref, wref, u)

    def wait_g(sref, xref, gsem):
        pltpu.make_async_copy(x_hbm.at[sref.at[0]], xref, gsem).wait()

    def scale(xref, wref):
        def row(g, _):
            wv = wref[pl.ds(g * 16, 16)]
            for jj in range(16):
                r = g * 16 + jj
                wr = wv[jj]
                for cc in range(8):
                    xref[r, pl.ds(cc * 16, 16)] = (
                        xref[r, pl.ds(cc * 16, 16)] * wr)
            return 0
        lax.fori_loop(0, HB // 16, row, 0)

    def scatter(xref, dref, ssem):
        pltpu.async_copy(xref, agg_sh.at[dref.at[0]], ssem, add=True)

    def wait_s(xref, dref, ssem):
        pltpu.make_async_copy(xref, agg_sh.at[dref.at[0]], ssem).wait()

    for hh in range(2):
        h = 2 * core + hh

        def zb(i, _):
            for cc in range(8):
                xrA[i, pl.ds(cc * 16, 16)] = jnp.zeros((16,), jnp.float32)
            return 0
        lax.fori_loop(0, HB, zb, 0)
        for k in range(ROWS_T // HB):
            pltpu.sync_copy(xrA, agg_sh.at[pl.ds(sub * ROWS_T + k * HB, HB)])

        def zd(i, _):
            denom[pl.ds(i * 16, 16)] = jnp.zeros((16,), jnp.float32)
            return 0
        lax.fori_loop(0, NP // 16, zd, 0)
        pltpu.sync_copy(a1T.at[h], tbl_s)
        pltpu.sync_copy(a1T.at[4 + h], tbl_d)
        plsc.subcore_barrier()

        stage_in(0, sA, dA, xrA, gsA, wA)

        def pair(i, _):
            u0 = 2 * i
            wait_g(sA, xrA, gsA)
            scale(xrA, wA)
            scatter(xrA, dA, ssA)

            @pl.when(i >= 1)
            def _():
                wait_s(xrB, dB, ssB)
            stage_in(u0 + 1, sB, dB, xrB, gsB, wB)
            wait_g(sB, xrB, gsB)
            scale(xrB, wB)
            scatter(xrB, dB, ssB)

            @pl.when(i + 1 < NQ // 2)
            def _():
                wait_s(xrA, dA, ssA)
                stage_in(u0 + 2, sA, dA, xrA, gsA, wA)
            return 0
        lax.fori_loop(0, NQ // 2, pair, 0)
        wait_s(xrA, dA, ssA)
        wait_s(xrB, dB, ssB)
        plsc.subcore_barrier()
        pltpu.sync_copy(agg_sh.at[pl.ds(sub * ROWS_T, ROWS_T)],
                        aggw.at[h, pl.ds(sub * ROWS_T, ROWS_T)])
        pltpu.sync_copy(denom, dpart.at[h, sub])
        plsc.subcore_barrier()


def _k3_body(aggw_ref, dinv_ref, gW1_ref, gb1_ref, gW2_ref, ga2c_ref,
             h2_ref, a2T_ref):
    j = pl.program_id(0)
    rmask = (j * 1024 + lax.broadcasted_iota(jnp.int32, (1024, 1), 0)) < N
    parts = []
    for h in range(H):
        ah = aggw_ref[h] * dinv_ref[:, h:h + 1]
        ph = lax.dot_general(ah, gW1_ref[:, h * C1:(h + 1) * C1],
                             (((1,), (0,)), ((), ())), precision=HI)
        ph = ph + gb1_ref[:, h * C1:(h + 1) * C1]
        parts.append(jnp.where(rmask, jnp.maximum(ph, 0.0), 0.0))
    h2 = jnp.concatenate(parts, axis=1)
    h2_ref[...] = h2
    usd = lax.dot_general(gW2_ref[...], ga2c_ref[...],
                          (((1,), (1,)), ((), ())), precision=HI)
    a2T_ref[...] = lax.dot_general(usd, h2, (((0,), (1,)), ((), ())),
                                   precision=HI)


def _k4_body(a2T, s_hbm, d_hbm, m_hbm, cpart, dsum,
             tbl_as, tbl_ad, cpriv, src_b, dst_b, mbuf, accbuf):
    core = lax.axis_index("c")
    sub = lax.axis_index("s")
    wid = core * NSUB + sub
    pltpu.sync_copy(a2T.at[0], tbl_as)
    pltpu.sync_copy(a2T.at[1], tbl_ad)
    pltpu.sync_copy(m_hbm, mbuf)
    mv = mbuf[...]
    ad2m = plsc.load_gather(tbl_ad, [mv])
    eself = _leaky(plsc.load_gather(tbl_as, [mv]) + ad2m)

    def zc(i, _):
        cpriv[pl.ds(i * 16, 16)] = jnp.zeros((16,), jnp.float32)
        return 0
    lax.fori_loop(0, NP // 16, zc, 0)

    def chunk(k, dacc):
        off = wid * PT_K4 + k * B_K4
        pltpu.sync_copy(s_hbm.at[pl.ds(off, B_K4)], src_b)
        pltpu.sync_copy(d_hbm.at[pl.ds(off, B_K4)], dst_b)

        def it(i, acc):
            s16 = src_b[pl.ds(i * 16, 16)]
            d16 = dst_b[pl.ds(i * 16, 16)]
            msk = d16 == mv
            a_s = plsc.load_gather(tbl_as, [s16])
            w16 = jnp.exp(_leaky(a_s + ad2m) - eself)
            plsc.addupdate_scatter(cpriv, [s16], w16, mask=msk)
            return acc + jnp.where(msk, w16, 0.0)
        return lax.fori_loop(0, B_K4 // 16, it, dacc)
    dacc = lax.fori_loop(0, PT_K4 // B_K4, chunk, jnp.zeros((16,),
                                                            jnp.float32))
    accbuf[...] = dacc
    pltpu.sync_copy(cpriv, cpart.at[wid])
    pltpu.sync_copy(accbuf, dsum.at[wid])


def _k5_body(c_ref, h2_ref, dinv2_ref, msd_ref, mW1_ref, mb1_ref, mW2_ref,
             mb2_ref, gW2_ref, gb2_ref, pW1_ref, pb1_ref, pW2_ref, pb2_ref,
             pW3_ref, pb3_ref, o_ref, racc):
    j = pl.program_id(0)

    @pl.when(j == 0)
    def _():
        racc[...] = jnp.zeros_like(racc)

    racc[...] += lax.dot_general(c_ref[...], h2_ref[...],
                                 (((1,), (0,)), ((), ())), precision=HI)

    @pl.when(j == pl.num_programs(0) - 1)
    def _():
        feat = jnp.maximum(
            lax.dot_general(racc[...], gW2_ref[...], (((1,), (0,)), ((), ())),
                            precision=HI) * dinv2_ref[0, 0] + gb2_ref[...],
            0.0)
        mut = jnp.maximum(jnp.dot(msd_ref[...], mW1_ref[...], precision=HI)
                          + mb1_ref[...], 0.0)
        mut = jnp.dot(mut, mW2_ref[...], precision=HI) + mb2_ref[...]
        z = jnp.concatenate([feat, mut], axis=1)
        z = jnp.maximum(jnp.dot(z, pW1_ref[...], precision=HI)
                        + pb1_ref[...], 0.0)
        z = jnp.maximum(jnp.dot(z, pW2_ref[...], precision=HI)
                        + pb2_ref[...], 0.0)
        o_ref[...] = jnp.dot(z, pW3_ref[...], precision=HI) + pb3_ref[...]


def kernel(x, edge_index, mutation_idx, mutation_site_diff, gW1, ga1_src,
           ga1_dst, gb1, gW2, ga2_src, ga2_dst, gb2, mW1, mb1, mW2, mb2,
           pW1, pb1, pW2, pb2, pW3, pb3):
    msd = mutation_site_diff
    if msd.ndim == 1:
        msd = msd[None, :]
    x_p = jnp.pad(x, ((0, NP - N), (0, 0)))
    loops = jnp.arange(N, dtype=edge_index.dtype)
    s_all = jnp.concatenate(
        [edge_index[0], loops,
         jnp.zeros((EP - ETOT,), edge_index.dtype)])
    d_all = jnp.concatenate(
        [edge_index[1], loops,
         jnp.zeros((EP - ETOT,), edge_index.dtype)])
    m_arr = jnp.full((16,), mutation_idx, jnp.int32)

    a1T = pl.pallas_call(
        _k1_body,
        grid=(NP // 1024,),
        in_specs=[
            pl.BlockSpec((1024, D), lambda j: (j, 0)),
            pl.BlockSpec((D, H * C1), lambda j: (0, 0)),
            pl.BlockSpec((H, C1), lambda j: (0, 0)),
            pl.BlockSpec((H, C1), lambda j: (0, 0)),
        ],
        out_specs=pl.BlockSpec((2 * H, 1024), lambda j: (0, j)),
        out_shape=jax.ShapeDtypeStruct((2 * H, NP), jnp.float32),
    )(x_p, gW1, ga1_src, ga1_dst)

    mesh = plsc.VectorSubcoreMesh(core_axis_name="c", subcore_axis_name="s")
    k2 = pl.kernel(
        _k2_body,
        out_type=[
            jax.ShapeDtypeStruct((H, NP, D), jnp.float32),
            jax.ShapeDtypeStruct((H, NSUB, NP), jnp.float32),
        ],
        mesh=mesh,
        scratch_types=[
            pltpu.VMEM_SHARED((NP, D), jnp.float32),
            pltpu.VMEM((NP,), jnp.float32),
            pltpu.VMEM((NP,), jnp.float32),
            pltpu.VMEM((NP,), jnp.float32),
            pltpu.VMEM((1, HB), jnp.int32),
            pltpu.VMEM((1, HB), jnp.int32),
            pltpu.VMEM((1, HB), jnp.int32),
            pltpu.VMEM((1, HB), jnp.int32),
            pltpu.VMEM((HB,), jnp.float32),
            pltpu.VMEM((HB,), jnp.float32),
            pltpu.VMEM((HB, D), jnp.float32),
            pltpu.VMEM((HB, D), jnp.float32),
            pltpu.SemaphoreType.DMA,
            pltpu.SemaphoreType.DMA,
            pltpu.SemaphoreType.DMA,
            pltpu.SemaphoreType.DMA,
        ],
        compiler_params=pltpu.CompilerParams(needs_layout_passes=False),
    )
    aggw, dpart = k2(a1T, x_p, s_all, d_all)

    dsum1 = jnp.sum(dpart, axis=1)
    dinvT = jnp.transpose(1.0 / (dsum1 + 1e-16))

    h2, a2T = pl.pallas_call(
        _k3_body,
        grid=(NP // 1024,),
        in_specs=[
            pl.BlockSpec((H, 1024, D), lambda j: (0, j, 0)),
            pl.BlockSpec((1024, H), lambda j: (j, 0)),
            pl.BlockSpec((D, H * C1), lambda j: (0, 0)),
            pl.BlockSpec((1, H * C1), lambda j: (0, 0)),
            pl.BlockSpec((H * C1, D), lambda j: (0, 0)),
            pl.BlockSpec((2, D), lambda j: (0, 0)),
        ],
        out_specs=[
            pl.BlockSpec((1024, H * C1), lambda j: (j, 0)),
            pl.BlockSpec((2, 1024), lambda j: (0, j)),
        ],
        out_shape=[
            jax.ShapeDtypeStruct((NP, H * C1), jnp.float32),
            jax.ShapeDtypeStruct((2, NP), jnp.float32),
        ],
    )(aggw, dinvT, gW1, gb1[None, :], gW2,
      jnp.concatenate([ga2_src, ga2_dst], axis=0))

    k4 = pl.kernel(
        _k4_body,
        out_type=[
            jax.ShapeDtypeStruct((2 * NSUB, NP), jnp.float32),
            jax.ShapeDtypeStruct((2 * NSUB, 16), jnp.float32),
        ],
        mesh=mesh,
        scratch_types=[
            pltpu.VMEM((NP,), jnp.float32),
            pltpu.VMEM((NP,), jnp.float32),
            pltpu.VMEM((NP,), jnp.float32),
            pltpu.VMEM((B_K4,), jnp.int32),
            pltpu.VMEM((B_K4,), jnp.int32),
            pltpu.VMEM((16,), jnp.int32),
            pltpu.VMEM((16,), jnp.float32),
        ],
        compiler_params=pltpu.CompilerParams(needs_layout_passes=False),
    )
    cpart, dsum2 = k4(a2T, s_all, d_all, m_arr)

    c = jnp.sum(cpart, axis=0)
    c = jnp.where(jnp.arange(NP) < N, c, 0.0)
    c = c.at[mutation_idx].add(1.0)
    denom2 = jnp.sum(dsum2) + 1.0
    dinv2 = (1.0 / (denom2 + 1e-16)).reshape(1, 1)

    out = pl.pallas_call(
        _k5_body,
        grid=(NP // 1024,),
        in_specs=[
            pl.BlockSpec((1, 1024), lambda j: (0, j)),
            pl.BlockSpec((1024, H * C1), lambda j: (j, 0)),
            pl.BlockSpec((1, 1), lambda j: (0, 0)),
            pl.BlockSpec((1, 1024), lambda j: (0, 0)),
            pl.BlockSpec((1024, 128), lambda j: (0, 0)),
            pl.BlockSpec((1, 128), lambda j: (0, 0)),
            pl.BlockSpec((128, 128), lambda j: (0, 0)),
            pl.BlockSpec((1, 128), lambda j: (0, 0)),
            pl.BlockSpec((H * C1, 128), lambda j: (0, 0)),
            pl.BlockSpec((1, 128), lambda j: (0, 0)),
            pl.BlockSpec((256, 128), lambda j: (0, 0)),
            pl.BlockSpec((1, 128), lambda j: (0, 0)),
            pl.BlockSpec((128, 32), lambda j: (0, 0)),
            pl.BlockSpec((1, 32), lambda j: (0, 0)),
            pl.BlockSpec((32, 1), lambda j: (0, 0)),
            pl.BlockSpec((1, 1), lambda j: (0, 0)),
        ],
        out_specs=pl.BlockSpec((1, 1), lambda j: (0, 0)),
        out_shape=jax.ShapeDtypeStruct((1, 1), jnp.float32),
        scratch_shapes=[pltpu.VMEM((1, H * C1), jnp.float32)],
    )(c[None, :], h2, dinv2, msd, mW1, mb1[None, :], mW2, mb2[None, :],
      gW2, gb2[None, :], pW1, pb1[None, :], pW2, pb2[None, :], pW3,
      pb3[None, :])
    return out


# speedup vs baseline: 1.2263x; 1.2263x over previous
import functools

import jax
from jax import lax
import jax.numpy as jnp
from jax.experimental import pallas as pl
from jax.experimental.pallas import tpu as pltpu
from jax.experimental.pallas import tpu_sc as plsc

N = 10000
NP = 10240
D = 128
H = 4
C1 = 256
E = 320000
ETOT = E + N
EB = 128
HB = 64
NSUB = 16
NB_K2 = -(-ETOT // (NSUB * EB))
PT_K2 = NB_K2 * EB
NQ = PT_K2 // HB
EP = NSUB * PT_K2
ROWS_T = NP // NSUB
PT_K4 = E // (2 * NSUB)
B_K4 = 1000
HI = lax.Precision.HIGHEST


def _leaky(t):
    return jnp.where(t > 0, t, 0.2 * t)


def _k1_body(x_ref, gW1_ref, gas_ref, gad_ref, o_ref):
    cols = []
    for tbl_ref in (gas_ref, gad_ref):
        for h in range(H):
            blk = gW1_ref[:, h * C1:(h + 1) * C1] * tbl_ref[h:h + 1, :]
            cols.append(jnp.sum(blk, axis=1, keepdims=True))
    vsd = jnp.concatenate(cols, axis=1)
    o_ref[...] = lax.dot_general(vsd, x_ref[...], (((0,), (1,)), ((), ())),
                                 precision=HI)


def _k2_body(a1T, x_hbm, e2_hbm, aggw, dpart, agg_sh,
             tbl_s, tbl_d, denom, ed_b, w_b, xrows, sem):
    core = lax.axis_index("c")
    sub = lax.axis_index("s")

    for hh in range(2):
        h = 2 * core + hh

        def zb(i, _):
            for cc in range(8):
                xrows[i, pl.ds(cc * 16, 16)] = jnp.zeros((16,), jnp.float32)
            return 0
        lax.fori_loop(0, EB, zb, 0)
        for k in range(ROWS_T // EB):
            pltpu.sync_copy(xrows, agg_sh.at[pl.ds(sub * ROWS_T + k * EB,
                                                   EB)])

        def zd(i, _):
            denom[pl.ds(i * 16, 16)] = jnp.zeros((16,), jnp.float32)
            return 0
        lax.fori_loop(0, NP // 16, zd, 0)
        pltpu.sync_copy(a1T.at[h], tbl_s)
        pltpu.sync_copy(a1T.at[4 + h], tbl_d)
        plsc.subcore_barrier()

        def blk(b, _):
            off = sub * PT_K2 + b * EB
            pltpu.sync_copy(e2_hbm.at[:, pl.ds(off, EB)], ed_b)
            cp = pltpu.async_copy(x_hbm.at[ed_b.at[0]], xrows, sem)
            for i in range(EB // 16):
                s16 = ed_b[0, pl.ds(i * 16, 16)]
                d16 = ed_b[1, pl.ds(i * 16, 16)]
                a_s = plsc.load_gather(tbl_s, [s16])
                a_d = plsc.load_gather(tbl_d, [d16])
                w16 = jnp.exp(_leaky(a_s + a_d) - a_d)
                eid = off + i * 16 + lax.iota(jnp.int32, 16)
                w16 = jnp.where(eid < ETOT, w16, 0.0)
                w_b[pl.ds(i * 16, 16)] = w16
                plsc.addupdate_scatter(denom, [d16], w16)
            cp.wait()

            def row(g, _):
                wv = w_b[pl.ds(g * 16, 16)]
                for jj in range(16):
                    r = g * 16 + jj
                    wr = wv[jj]
                    for cc in range(8):
                        xrows[r, pl.ds(cc * 16, 16)] = (
                            xrows[r, pl.ds(cc * 16, 16)] * wr)
                return 0
            lax.fori_loop(0, EB // 16, row, 0)
            pltpu.sync_copy(xrows, agg_sh.at[ed_b.at[1]], add=True)
            return 0
        lax.fori_loop(0, NB_K2, blk, 0)
        plsc.subcore_barrier()
        pltpu.sync_copy(agg_sh.at[pl.ds(sub * ROWS_T, ROWS_T)],
                        aggw.at[h, pl.ds(sub * ROWS_T, ROWS_T)])
        pltpu.sync_copy(denom, dpart.at[h, sub])
        plsc.subcore_barrier()


def _k3_body(aggw_ref, dinv_ref, gW1_ref, gb1_ref, gW2_ref, ga2c_ref,
             h2_ref, a2T_ref):
    j = pl.program_id(0)
    rmask = (j * 1024 + lax.broadcasted_iota(jnp.int32, (1024, 1), 0)) < N
    parts = []
    for h in range(H):
        ah = aggw_ref[h] * dinv_ref[:, h:h + 1]
        ph = lax.dot_general(ah, gW1_ref[:, h * C1:(h + 1) * C1],
                             (((1,), (0,)), ((), ())), precision=HI)
        ph = ph + gb1_ref[:, h * C1:(h + 1) * C1]
        parts.append(jnp.where(rmask, jnp.maximum(ph, 0.0), 0.0))
    h2 = jnp.concatenate(parts, axis=1)
    h2_ref[...] = h2
    usd = lax.dot_general(gW2_ref[...], ga2c_ref[...],
                          (((1,), (1,)), ((), ())), precision=HI)
    a2T_ref[...] = lax.dot_general(usd, h2, (((0,), (1,)), ((), ())),
                                   precision=HI)


def _k4_body(a2T, s_hbm, d_hbm, m_hbm, cpart, dsum,
             tbl_as, tbl_ad, cpriv, src_b, dst_b, mbuf, accbuf):
    core = lax.axis_index("c")
    sub = lax.axis_index("s")
    wid = core * NSUB + sub
    pltpu.sync_copy(a2T.at[0], tbl_as)
    pltpu.sync_copy(a2T.at[1], tbl_ad)
    pltpu.sync_copy(m_hbm, mbuf)
    mv = mbuf[...]
    ad2m = plsc.load_gather(tbl_ad, [mv])
    eself = _leaky(plsc.load_gather(tbl_as, [mv]) + ad2m)

    def zc(i, _):
        cpriv[pl.ds(i * 16, 16)] = jnp.zeros((16,), jnp.float32)
        return 0
    lax.fori_loop(0, NP // 16, zc, 0)

    def chunk(k, dacc):
        off = wid * PT_K4 + k * B_K4
        pltpu.sync_copy(s_hbm.at[pl.ds(off, B_K4)], src_b)
        pltpu.sync_copy(d_hbm.at[pl.ds(off, B_K4)], dst_b)

        def it(i, acc):
            s16 = src_b[pl.ds(i * 16, 16)]
            d16 = dst_b[pl.ds(i * 16, 16)]
            msk = d16 == mv
            a_s = plsc.load_gather(tbl_as, [s16])
            w16 = jnp.exp(_leaky(a_s + ad2m) - eself)
            plsc.addupdate_scatter(cpriv, [s16], w16, mask=msk)
            return acc + jnp.where(msk, w16, 0.0)
        return lax.fori_loop(0, B_K4 // 16, it, dacc)
    dacc = lax.fori_loop(0, PT_K4 // B_K4, chunk, jnp.zeros((16,),
                                                            jnp.float32))
    accbuf[...] = dacc
    pltpu.sync_copy(cpriv, cpart.at[wid])
    pltpu.sync_copy(accbuf, dsum.at[wid])


def _k5_body(c_ref, h2_ref, dinv2_ref, msd_ref, mW1_ref, mb1_ref, mW2_ref,
             mb2_ref, gW2_ref, gb2_ref, pW1_ref, pb1_ref, pW2_ref, pb2_ref,
             pW3_ref, pb3_ref, o_ref, racc):
    j = pl.program_id(0)

    @pl.when(j == 0)
    def _():
        racc[...] = jnp.zeros_like(racc)

    racc[...] += lax.dot_general(c_ref[...], h2_ref[...],
                                 (((1,), (0,)), ((), ())), precision=HI)

    @pl.when(j == pl.num_programs(0) - 1)
    def _():
        feat = jnp.maximum(
            lax.dot_general(racc[...], gW2_ref[...], (((1,), (0,)), ((), ())),
                            precision=HI) * dinv2_ref[0, 0] + gb2_ref[...],
            0.0)
        mut = jnp.maximum(jnp.dot(msd_ref[...], mW1_ref[...], precision=HI)
                          + mb1_ref[...], 0.0)
        mut = jnp.dot(mut, mW2_ref[...], precision=HI) + mb2_ref[...]
        z = jnp.concatenate([feat, mut], axis=1)
        z = jnp.maximum(jnp.dot(z, pW1_ref[...], precision=HI)
                        + pb1_ref[...], 0.0)
        z = jnp.maximum(jnp.dot(z, pW2_ref[...], precision=HI)
                        + pb2_ref[...], 0.0)
        o_ref[...] = jnp.dot(z, pW3_ref[...], precision=HI) + pb3_ref[...]


def kernel(x, edge_index, mutation_idx, mutation_site_diff, gW1, ga1_src,
           ga1_dst, gb1, gW2, ga2_src, ga2_dst, gb2, mW1, mb1, mW2, mb2,
           pW1, pb1, pW2, pb2, pW3, pb3):
    msd = mutation_site_diff
    if msd.ndim == 1:
        msd = msd[None, :]
    x_p = jnp.pad(x, ((0, NP - N), (0, 0)))
    loops = jnp.arange(N, dtype=edge_index.dtype)
    s_all = jnp.concatenate(
        [edge_index[0], loops,
         jnp.zeros((EP - ETOT,), edge_index.dtype)])
    d_all = jnp.concatenate(
        [edge_index[1], loops,
         jnp.zeros((EP - ETOT,), edge_index.dtype)])
    m_arr = jnp.full((16,), mutation_idx, jnp.int32)

    a1T = pl.pallas_call(
        _k1_body,
        grid=(NP // 1024,),
        in_specs=[
            pl.BlockSpec((1024, D), lambda j: (j, 0)),
            pl.BlockSpec((D, H * C1), lambda j: (0, 0)),
            pl.BlockSpec((H, C1), lambda j: (0, 0)),
            pl.BlockSpec((H, C1), lambda j: (0, 0)),
        ],
        out_specs=pl.BlockSpec((2 * H, 1024), lambda j: (0, j)),
        out_shape=jax.ShapeDtypeStruct((2 * H, NP), jnp.float32),
    )(x_p, gW1, ga1_src, ga1_dst)

    mesh = plsc.VectorSubcoreMesh(core_axis_name="c", subcore_axis_name="s")
    k2 = pl.kernel(
        _k2_body,
        out_type=[
            jax.ShapeDtypeStruct((H, NP, D), jnp.float32),
            jax.ShapeDtypeStruct((H, NSUB, NP), jnp.float32),
        ],
        mesh=mesh,
        scratch_types=[
            pltpu.VMEM_SHARED((NP, D), jnp.float32),
            pltpu.VMEM((NP,), jnp.float32),
            pltpu.VMEM((NP,), jnp.float32),
            pltpu.VMEM((NP,), jnp.float32),
            pltpu.VMEM((2, EB), jnp.int32),
            pltpu.VMEM((EB,), jnp.float32),
            pltpu.VMEM((EB, D), jnp.float32),
            pltpu.SemaphoreType.DMA,
        ],
        compiler_params=pltpu.CompilerParams(needs_layout_passes=False),
    )
    e2_all = jnp.stack([s_all, d_all])
    aggw, dpart = k2(a1T, x_p, e2_all)

    dsum1 = jnp.sum(dpart, axis=1)
    dinvT = jnp.transpose(1.0 / (dsum1 + 1e-16))

    h2, a2T = pl.pallas_call(
        _k3_body,
        grid=(NP // 1024,),
        in_specs=[
            pl.BlockSpec((H, 1024, D), lambda j: (0, j, 0)),
            pl.BlockSpec((1024, H), lambda j: (j, 0)),
            pl.BlockSpec((D, H * C1), lambda j: (0, 0)),
            pl.BlockSpec((1, H * C1), lambda j: (0, 0)),
            pl.BlockSpec((H * C1, D), lambda j: (0, 0)),
            pl.BlockSpec((2, D), lambda j: (0, 0)),
        ],
        out_specs=[
            pl.BlockSpec((1024, H * C1), lambda j: (j, 0)),
            pl.BlockSpec((2, 1024), lambda j: (0, j)),
        ],
        out_shape=[
            jax.ShapeDtypeStruct((NP, H * C1), jnp.float32),
            jax.ShapeDtypeStruct((2, NP), jnp.float32),
        ],
    )(aggw, dinvT, gW1, gb1[None, :], gW2,
      jnp.concatenate([ga2_src, ga2_dst], axis=0))

    k4 = pl.kernel(
        _k4_body,
        out_type=[
            jax.ShapeDtypeStruct((2 * NSUB, NP), jnp.float32),
            jax.ShapeDtypeStruct((2 * NSUB, 16), jnp.float32),
        ],
        mesh=mesh,
        scratch_types=[
            pltpu.VMEM((NP,), jnp.float32),
            pltpu.VMEM((NP,), jnp.float32),
            pltpu.VMEM((NP,), jnp.float32),
            pltpu.VMEM((B_K4,), jnp.int32),
            pltpu.VMEM((B_K4,), jnp.int32),
            pltpu.VMEM((16,), jnp.int32),
            pltpu.VMEM((16,), jnp.float32),
        ],
        compiler_params=pltpu.CompilerParams(needs_layout_passes=False),
    )
    cpart, dsum2 = k4(a2T, s_all, d_all, m_arr)

    c = jnp.sum(cpart, axis=0)
    c = jnp.where(jnp.arange(NP) < N, c, 0.0)
    c = c.at[mutation_idx].add(1.0)
    denom2 = jnp.sum(dsum2) + 1.0
    dinv2 = (1.0 / (denom2 + 1e-16)).reshape(1, 1)

    out = pl.pallas_call(
        _k5_body,
        grid=(NP // 1024,),
        in_specs=[
            pl.BlockSpec((1, 1024), lambda j: (0, j)),
            pl.BlockSpec((1024, H * C1), lambda j: (j, 0)),
            pl.BlockSpec((1, 1), lambda j: (0, 0)),
            pl.BlockSpec((1, 1024), lambda j: (0, 0)),
            pl.BlockSpec((1024, 128), lambda j: (0, 0)),
            pl.BlockSpec((1, 128), lambda j: (0, 0)),
            pl.BlockSpec((128, 128), lambda j: (0, 0)),
            pl.BlockSpec((1, 128), lambda j: (0, 0)),
            pl.BlockSpec((H * C1, 128), lambda j: (0, 0)),
            pl.BlockSpec((1, 128), lambda j: (0, 0)),
            pl.BlockSpec((256, 128), lambda j: (0, 0)),
            pl.BlockSpec((1, 128), lambda j: (0, 0)),
            pl.BlockSpec((128, 32), lambda j: (0, 0)),
            pl.BlockSpec((1, 32), lambda j: (0, 0)),
            pl.BlockSpec((32, 1), lambda j: (0, 0)),
            pl.BlockSpec((1, 1), lambda j: (0, 0)),
        ],
        out_specs=pl.BlockSpec((1, 1), lambda j: (0, 0)),
        out_shape=jax.ShapeDtypeStruct((1, 1), jnp.float32),
        scratch_shapes=[pltpu.VMEM((1, H * C1), jnp.float32)],
    )(c[None, :], h2, dinv2, msd, mW1, mb1[None, :], mW2, mb2[None, :],
      gW2, gb2[None, :], pW1, pb1[None, :], pW2, pb2[None, :], pW3,
      pb3[None, :])
    return out


# speedup vs baseline: 1.3833x; 1.1281x over previous
import functools

import jax
from jax import lax
import jax.numpy as jnp
from jax.experimental import pallas as pl
from jax.experimental.pallas import tpu as pltpu
from jax.experimental.pallas import tpu_sc as plsc

N = 10000
NP = 10240
D = 128
H = 4
C1 = 256
E = 320000
ETOT = E + N
EB = 128
HB = 64
NSUB = 16
NB_K2 = -(-ETOT // (NSUB * EB))
PT_K2 = NB_K2 * EB
NQ = PT_K2 // HB
EP = NSUB * PT_K2
ROWS_T = NP // NSUB
PT_K4 = E // (2 * NSUB)
B_K4 = 1000
HI = lax.Precision.HIGHEST


def _leaky(t):
    return jnp.where(t > 0, t, 0.2 * t)


def _k1_body(x_ref, gW1_ref, gas_ref, gad_ref, o_ref):
    hblk = jnp.dot(x_ref[...], gW1_ref[...])
    cols = []
    for tbl_ref in (gas_ref, gad_ref):
        for h in range(H):
            blk = hblk[:, h * C1:(h + 1) * C1] * tbl_ref[h:h + 1, :]
            cols.append(jnp.sum(blk, axis=1, keepdims=True))
    a = jnp.concatenate(cols, axis=1)
    eye = (lax.broadcasted_iota(jnp.int32, (2 * H, 2 * H), 0)
           == lax.broadcasted_iota(jnp.int32, (2 * H, 2 * H), 1)
           ).astype(jnp.float32)
    o_ref[...] = lax.dot_general(eye, a, (((1,), (1,)), ((), ())),
                                 precision=HI)


def _k2_body(a1T, x_hbm, e2_hbm, aggw, dpart, agg_sh,
             tbl_s, tbl_d, denom, edA, edB, wA, wB, xrows, gsem, ssem):
    core = lax.axis_index("c")
    sub = lax.axis_index("s")

    def scalar_phase(ed, wref, u):
        base = sub * PT_K2 + u * EB
        pltpu.sync_copy(e2_hbm.at[:, pl.ds(base, EB)], ed)
        for i in range(EB // 16):
            s16 = ed[0, pl.ds(i * 16, 16)]
            d16 = ed[1, pl.ds(i * 16, 16)]
            a_s = plsc.load_gather(tbl_s, [s16])
            a_d = plsc.load_gather(tbl_d, [d16])
            w16 = jnp.exp(_leaky(a_s + a_d) - a_d)
            eid = base + i * 16 + lax.iota(jnp.int32, 16)
            w16 = jnp.where(eid < ETOT, w16, 0.0)
            wref[pl.ds(i * 16, 16)] = w16
            plsc.addupdate_scatter(denom, [d16], w16)

    def scale(wref):
        def row(g, _):
            wv = wref[pl.ds(g * 16, 16)]
            for jj in range(16):
                r = g * 16 + jj
                wr = wv[jj]
                for cc in range(8):
                    xrows[r, pl.ds(cc * 16, 16)] = (
                        xrows[r, pl.ds(cc * 16, 16)] * wr)
            return 0
        lax.fori_loop(0, EB // 16, row, 0)

    def blockstep(u, edX, wX, edY, wY):
        pltpu.make_async_copy(x_hbm.at[edX.at[0]], xrows, gsem).wait()
        scale(wX)
        pltpu.async_copy(xrows, agg_sh.at[edX.at[1]], ssem, add=True)

        @pl.when(u + 1 < NB_K2)
        def _():
            scalar_phase(edY, wY, u + 1)
        pltpu.make_async_copy(xrows, agg_sh.at[edX.at[1]], ssem).wait()

        @pl.when(u + 1 < NB_K2)
        def _():
            pltpu.async_copy(x_hbm.at[edY.at[0]], xrows, gsem)

    for hh in range(2):
        h = 2 * core + hh

        def zb(i, _):
            for cc in range(8):
                xrows[i, pl.ds(cc * 16, 16)] = jnp.zeros((16,), jnp.float32)
            return 0
        lax.fori_loop(0, EB, zb, 0)
        for k in range(ROWS_T // EB):
            pltpu.sync_copy(xrows, agg_sh.at[pl.ds(sub * ROWS_T + k * EB,
                                                   EB)])

        def zd(i, _):
            denom[pl.ds(i * 16, 16)] = jnp.zeros((16,), jnp.float32)
            return 0
        lax.fori_loop(0, NP // 16, zd, 0)
        pltpu.sync_copy(a1T.at[h], tbl_s)
        pltpu.sync_copy(a1T.at[4 + h], tbl_d)
        plsc.subcore_barrier()

        scalar_phase(edA, wA, 0)
        pltpu.async_copy(x_hbm.at[edA.at[0]], xrows, gsem)

        def pair(i, _):
            blockstep(2 * i, edA, wA, edB, wB)
            blockstep(2 * i + 1, edB, wB, edA, wA)
            return 0
        lax.fori_loop(0, NB_K2 // 2, pair, 0)
        plsc.subcore_barrier()
        pltpu.sync_copy(agg_sh.at[pl.ds(sub * ROWS_T, ROWS_T)],
                        aggw.at[h, pl.ds(sub * ROWS_T, ROWS_T)])
        pltpu.sync_copy(denom, dpart.at[h, sub])
        plsc.subcore_barrier()


def _k3_body(aggw_ref, dinv_ref, gW1_ref, gb1_ref, gW2_ref, ga2c_ref,
             h2_ref, a2T_ref, g2_ref):
    j = pl.program_id(0)
    rmask = (j * 1024 + lax.broadcasted_iota(jnp.int32, (1024, 1), 0)) < N
    parts = []
    for h in range(H):
        ah = aggw_ref[h] * dinv_ref[:, h:h + 1]
        ph = lax.dot_general(ah, gW1_ref[:, h * C1:(h + 1) * C1],
                             (((1,), (0,)), ((), ())), precision=HI)
        ph = ph + gb1_ref[:, h * C1:(h + 1) * C1]
        parts.append(jnp.where(rmask, jnp.maximum(ph, 0.0), 0.0))
    h2 = jnp.concatenate(parts, axis=1)
    h2_ref[...] = h2
    g2 = jnp.dot(h2, gW2_ref[...])
    g2_ref[...] = g2
    a2 = lax.dot_general(ga2c_ref[...], g2, (((1,), (1,)), ((), ())),
                         precision=HI)
    a2T_ref[...] = a2


def _k4_body(a2T, s_hbm, d_hbm, m_hbm, cpart, dsum,
             tbl_as, tbl_ad, cpriv, src_b, dst_b, mbuf, accbuf):
    core = lax.axis_index("c")
    sub = lax.axis_index("s")
    wid = core * NSUB + sub
    pltpu.sync_copy(a2T.at[0], tbl_as)
    pltpu.sync_copy(a2T.at[1], tbl_ad)
    pltpu.sync_copy(m_hbm, mbuf)
    mv = mbuf[...]
    ad2m = plsc.load_gather(tbl_ad, [mv])
    eself = _leaky(plsc.load_gather(tbl_as, [mv]) + ad2m)

    def zc(i, _):
        cpriv[pl.ds(i * 16, 16)] = jnp.zeros((16,), jnp.float32)
        return 0
    lax.fori_loop(0, NP // 16, zc, 0)

    def chunk(k, dacc):
        off = wid * PT_K4 + k * B_K4
        pltpu.sync_copy(s_hbm.at[pl.ds(off, B_K4)], src_b)
        pltpu.sync_copy(d_hbm.at[pl.ds(off, B_K4)], dst_b)

        def it(i, acc):
            s16 = src_b[pl.ds(i * 16, 16)]
            d16 = dst_b[pl.ds(i * 16, 16)]
            msk = d16 == mv
            a_s = plsc.load_gather(tbl_as, [s16])
            w16 = jnp.exp(_leaky(a_s + ad2m) - eself)
            plsc.addupdate_scatter(cpriv, [s16], w16, mask=msk)
            return acc + jnp.where(msk, w16, 0.0)
        return lax.fori_loop(0, B_K4 // 16, it, dacc)
    dacc = lax.fori_loop(0, PT_K4 // B_K4, chunk, jnp.zeros((16,),
                                                            jnp.float32))
    accbuf[...] = dacc
    pltpu.sync_copy(cpriv, cpart.at[wid])
    pltpu.sync_copy(accbuf, dsum.at[wid])


def _k5_body(c_ref, g2_ref, dinv2_ref, msd_ref, mW1_ref, mb1_ref, mW2_ref,
             mb2_ref, gb2_ref, pW1_ref, pb1_ref, pW2_ref, pb2_ref,
             pW3_ref, pb3_ref, o_ref, racc):
    j = pl.program_id(0)

    @pl.when(j == 0)
    def _():
        racc[...] = jnp.zeros_like(racc)

    racc[...] += lax.dot_general(c_ref[...], g2_ref[...],
                                 (((1,), (0,)), ((), ())), precision=HI)

    @pl.when(j == pl.num_programs(0) - 1)
    def _():
        feat = jnp.maximum(racc[...] * dinv2_ref[0, 0] + gb2_ref[...], 0.0)
        mut = jnp.maximum(jnp.dot(msd_ref[...], mW1_ref[...])
                          + mb1_ref[...], 0.0)
        mut = jnp.dot(mut, mW2_ref[...]) + mb2_ref[...]
        z = jnp.concatenate([feat, mut], axis=1)
        z = jnp.maximum(jnp.dot(z, pW1_ref[...]) + pb1_ref[...], 0.0)
        z = jnp.maximum(jnp.dot(z, pW2_ref[...]) + pb2_ref[...], 0.0)
        o_ref[...] = jnp.dot(z, pW3_ref[...]) + pb3_ref[...]


def kernel(x, edge_index, mutation_idx, mutation_site_diff, gW1, ga1_src,
           ga1_dst, gb1, gW2, ga2_src, ga2_dst, gb2, mW1, mb1, mW2, mb2,
           pW1, pb1, pW2, pb2, pW3, pb3):
    msd = mutation_site_diff
    if msd.ndim == 1:
        msd = msd[None, :]
    x_p = jnp.pad(x.astype(jnp.bfloat16).astype(jnp.float32),
                  ((0, NP - N), (0, 0)))
    gW1_r = gW1.astype(jnp.bfloat16).astype(jnp.float32)
    loops = jnp.arange(N, dtype=edge_index.dtype)
    s_all = jnp.concatenate(
        [edge_index[0], loops,
         jnp.zeros((EP - ETOT,), edge_index.dtype)])
    d_all = jnp.concatenate(
        [edge_index[1], loops,
         jnp.zeros((EP - ETOT,), edge_index.dtype)])
    m_arr = jnp.full((16,), mutation_idx, jnp.int32)

    a1T = pl.pallas_call(
        _k1_body,
        grid=(NP // 1024,),
        in_specs=[
            pl.BlockSpec((1024, D), lambda j: (j, 0)),
            pl.BlockSpec((D, H * C1), lambda j: (0, 0)),
            pl.BlockSpec((H, C1), lambda j: (0, 0)),
            pl.BlockSpec((H, C1), lambda j: (0, 0)),
        ],
        out_specs=pl.BlockSpec((2 * H, 1024), lambda j: (0, j)),
        out_shape=jax.ShapeDtypeStruct((2 * H, NP), jnp.float32),
    )(x_p, gW1, ga1_src, ga1_dst)

    mesh = plsc.VectorSubcoreMesh(core_axis_name="c", subcore_axis_name="s")
    k2 = pl.kernel(
        _k2_body,
        out_type=[
            jax.ShapeDtypeStruct((H, NP, D), jnp.float32),
            jax.ShapeDtypeStruct((H, NSUB, NP), jnp.float32),
        ],
        mesh=mesh,
        scratch_types=[
            pltpu.VMEM_SHARED((NP, D), jnp.float32),
            pltpu.VMEM((NP,), jnp.float32),
            pltpu.VMEM((NP,), jnp.float32),
            pltpu.VMEM((NP,), jnp.float32),
            pltpu.VMEM((2, EB), jnp.int32),
            pltpu.VMEM((2, EB), jnp.int32),
            pltpu.VMEM((EB,), jnp.float32),
            pltpu.VMEM((EB,), jnp.float32),
            pltpu.VMEM((EB, D), jnp.float32),
            pltpu.SemaphoreType.DMA,
            pltpu.SemaphoreType.DMA,
        ],
        compiler_params=pltpu.CompilerParams(needs_layout_passes=False),
    )
    e2_all = jnp.stack([s_all, d_all])
    aggw, dpart = k2(a1T, x_p, e2_all)

    dsum1 = jnp.sum(dpart, axis=1)
    dinvT = jnp.transpose(1.0 / (dsum1 + 1e-16))

    h2, a2T, g2 = pl.pallas_call(
        _k3_body,
        grid=(NP // 1024,),
        in_specs=[
            pl.BlockSpec((H, 1024, D), lambda j: (0, j, 0)),
            pl.BlockSpec((1024, H), lambda j: (j, 0)),
            pl.BlockSpec((D, H * C1), lambda j: (0, 0)),
            pl.BlockSpec((1, H * C1), lambda j: (0, 0)),
            pl.BlockSpec((H * C1, D), lambda j: (0, 0)),
            pl.BlockSpec((2, D), lambda j: (0, 0)),
        ],
        out_specs=[
            pl.BlockSpec((1024, H * C1), lambda j: (j, 0)),
            pl.BlockSpec((2, 1024), lambda j: (0, j)),
            pl.BlockSpec((1024, D), lambda j: (j, 0)),
        ],
        out_shape=[
            jax.ShapeDtypeStruct((NP, H * C1), jnp.float32),
            jax.ShapeDtypeStruct((2, NP), jnp.float32),
            jax.ShapeDtypeStruct((NP, D), jnp.float32),
        ],
    )(aggw, dinvT, gW1_r, gb1[None, :], gW2,
      jnp.concatenate([ga2_src, ga2_dst], axis=0))

    k4 = pl.kernel(
        _k4_body,
        out_type=[
            jax.ShapeDtypeStruct((2 * NSUB, NP), jnp.float32),
            jax.ShapeDtypeStruct((2 * NSUB, 16), jnp.float32),
        ],
        mesh=mesh,
        scratch_types=[
            pltpu.VMEM((NP,), jnp.float32),
            pltpu.VMEM((NP,), jnp.float32),
            pltpu.VMEM((NP,), jnp.float32),
            pltpu.VMEM((B_K4,), jnp.int32),
            pltpu.VMEM((B_K4,), jnp.int32),
            pltpu.VMEM((16,), jnp.int32),
            pltpu.VMEM((16,), jnp.float32),
        ],
        compiler_params=pltpu.CompilerParams(needs_layout_passes=False),
    )
    cpart, dsum2 = k4(a2T, s_all, d_all, m_arr)

    c = jnp.sum(cpart, axis=0)
    c = jnp.where(jnp.arange(NP) < N, c, 0.0)
    c = c.at[mutation_idx].add(1.0)
    denom2 = jnp.sum(dsum2) + 1.0
    dinv2 = (1.0 / (denom2 + 1e-16)).reshape(1, 1)

    out = pl.pallas_call(
        _k5_body,
        grid=(NP // 1024,),
        in_specs=[
            pl.BlockSpec((1, 1024), lambda j: (0, j)),
            pl.BlockSpec((1024, D), lambda j: (j, 0)),
            pl.BlockSpec((1, 1), lambda j: (0, 0)),
            pl.BlockSpec((1, 1024), lambda j: (0, 0)),
            pl.BlockSpec((1024, 128), lambda j: (0, 0)),
            pl.BlockSpec((1, 128), lambda j: (0, 0)),
            pl.BlockSpec((128, 128), lambda j: (0, 0)),
            pl.BlockSpec((1, 128), lambda j: (0, 0)),
            pl.BlockSpec((1, 128), lambda j: (0, 0)),
            pl.BlockSpec((256, 128), lambda j: (0, 0)),
            pl.BlockSpec((1, 128), lambda j: (0, 0)),
            pl.BlockSpec((128, 32), lambda j: (0, 0)),
            pl.BlockSpec((1, 32), lambda j: (0, 0)),
            pl.BlockSpec((32, 1), lambda j: (0, 0)),
            pl.BlockSpec((1, 1), lambda j: (0, 0)),
        ],
        out_specs=pl.BlockSpec((1, 1), lambda j: (0, 0)),
        out_shape=jax.ShapeDtypeStruct((1, 1), jnp.float32),
        scratch_shapes=[pltpu.VMEM((1, D), jnp.float32)],
    )(c[None, :], g2, dinv2, msd, mW1, mb1[None, :], mW2, mb2[None, :],
      gb2[None, :], pW1, pb1[None, :], pW2, pb2[None, :], pW3,
      pb3[None, :])
    return out
